# baseline (speedup 1.0000x reference)
import functools

import jax
import jax.numpy as jnp
from jax.experimental import pallas as pl
from jax.experimental.pallas import tpu as pltpu

K = 20
A = 21
U = 64
EPS = 1e-06
AP = 32
GK = 8
NG = 3


def _r_kernel(p_ref, q_ref, r_ref, rt_ref):
    p = p_ref[...]
    m = jnp.max(p, axis=1, keepdims=True)
    e = jnp.exp(p - m)
    sm = e / jnp.sum(e, axis=1, keepdims=True)
    q = q_ref[0, :].reshape(1, A, 1)
    r = jnp.log(jnp.maximum(sm / q, EPS))
    r_ref[...] = r

    rb = r.astype(jnp.bfloat16)
    zcol = jnp.zeros((U, AP - A), jnp.bfloat16)
    for g in range(NG):
        nk = min(GK, K - GK * g)
        pieces = []
        for i in range(nk):
            rkt = rb[GK * g + i].T
            pieces.append(jnp.concatenate([rkt, zcol], axis=1))
        if nk < GK:
            pieces.append(jnp.zeros((U, AP * (GK - nk)), jnp.bfloat16))
        rt_ref[g] = jnp.concatenate(pieces, axis=1)


def _conv_kernel(x_ref, rt_ref, zt_ref, s_ref, *, pdim, n_batch, lanes, tiles):
    zrow = jnp.zeros((AP - A, lanes), jnp.bfloat16)
    seg = lanes // n_batch
    for tl in range(tiles):
        xt = x_ref[tl]
        acc = jnp.zeros((U, lanes), jnp.float32)
        for g in range(NG):
            nk = min(GK, K - GK * g)
            pieces = []
            for i in range(nk):
                k = GK * g + i
                rolled = pltpu.roll(xt, lanes - k, axis=1) if k else xt
                pieces.append(jnp.concatenate([rolled, zrow], axis=0))
            rhs = jnp.concatenate(pieces, axis=0)
            lhs = rt_ref[g]
            if nk < GK:
                lhs = jax.lax.slice_in_dim(lhs, 0, AP * nk, axis=1)
            acc = acc + jax.lax.dot(lhs, rhs,
                                    preferred_element_type=jnp.float32)
        for n in range(n_batch):
            blk = jax.lax.slice_in_dim(acc, seg * n, seg * n + pdim, axis=1)
            zt_ref[0, n, tl] = blk
            s_ref[0, tl, n] = jnp.max(blk, axis=1)


@jax.jit
def kernel(X, P_logit, Q):
    T, N, F, L, A_ = X.shape
    pdim = L - K + 1

    R, RT = pl.pallas_call(
        _r_kernel,
        out_shape=[
            jax.ShapeDtypeStruct((K, A, U), jnp.float32),
            jax.ShapeDtypeStruct((NG, U, AP * GK), jnp.bfloat16),
        ],
    )(P_logit, Q.reshape(1, A))

    TILES = 2
    Xp = X.transpose(0, 2, 4, 1, 3).reshape(T * F, A_, N * L)
    Xp = Xp.astype(jnp.bfloat16)
    FG = F // TILES

    Zt, Sp = pl.pallas_call(
        functools.partial(_conv_kernel, pdim=pdim, n_batch=N, lanes=N * L,
                          tiles=TILES),
        grid=(T * F // TILES,),
        in_specs=[
            pl.BlockSpec((TILES, A_, N * L), lambda b: (b, 0, 0)),
            pl.BlockSpec((NG, U, AP * GK), lambda b: (0, 0, 0)),
        ],
        out_specs=[
            pl.BlockSpec((1, N, TILES, U, pdim),
                         lambda b: (b // FG, 0, b % FG, 0, 0)),
            pl.BlockSpec((1, TILES, N, U), lambda b: (b // FG, b % FG, 0, 0)),
        ],
        out_shape=[
            jax.ShapeDtypeStruct((T, N, F, U, pdim), jnp.float32),
            jax.ShapeDtypeStruct((T, F, N, U), jnp.float32),
        ],
    )(Xp, RT)

    return (R, Sp.transpose(0, 2, 1, 3), jnp.swapaxes(Zt, 3, 4))


# speedup vs baseline: 1.0208x; 1.0208x over previous
import functools

import jax
import jax.numpy as jnp
from jax.experimental import pallas as pl
from jax.experimental.pallas import tpu as pltpu

K = 20
A = 21
U = 64
EPS = 1e-06
AP = 32
GK = 8
NG = 3


def _conv_kernel(p_ref, q_ref, x_ref, r_ref, zt_ref, s_ref, rt_ref,
                 *, pdim, n_batch, lanes):
    @pl.when(pl.program_id(0) == 0)
    def _():
        p = p_ref[...]
        m = jnp.max(p, axis=1, keepdims=True)
        e = jnp.exp(p - m)
        sm = e / jnp.sum(e, axis=1, keepdims=True)
        q = q_ref[0, :].reshape(1, A, 1)
        r = jnp.log(jnp.maximum(sm / q, EPS))
        r_ref[...] = r

        rb = r.astype(jnp.bfloat16)
        zcol = jnp.zeros((U, AP - A), jnp.bfloat16)
        for g in range(NG):
            nk = min(GK, K - GK * g)
            pieces = []
            for i in range(nk):
                rkt = rb[GK * g + i].T
                pieces.append(jnp.concatenate([rkt, zcol], axis=1))
            if nk < GK:
                pieces.append(jnp.zeros((U, AP * (GK - nk)), jnp.bfloat16))
            rt_ref[g] = jnp.concatenate(pieces, axis=1)

    xt = x_ref[0].astype(jnp.bfloat16)
    zrow = jnp.zeros((AP - A, lanes), jnp.bfloat16)
    acc = jnp.zeros((U, lanes), jnp.float32)
    for g in range(NG):
        nk = min(GK, K - GK * g)
        pieces = []
        for i in range(nk):
            k = GK * g + i
            rolled = pltpu.roll(xt, lanes - k, axis=1) if k else xt
            pieces.append(jnp.concatenate([rolled, zrow], axis=0))
        rhs = jnp.concatenate(pieces, axis=0)
        lhs = rt_ref[g]
        if nk < GK:
            lhs = jax.lax.slice_in_dim(lhs, 0, AP * nk, axis=1)
        acc = acc + jax.lax.dot(lhs, rhs, preferred_element_type=jnp.float32)
    seg = lanes // n_batch
    for n in range(n_batch):
        blk = jax.lax.slice_in_dim(acc, seg * n, seg * n + pdim, axis=1)
        zt_ref[0, n, 0] = blk
        s_ref[0, 0, n] = jnp.max(blk, axis=1)


@jax.jit
def kernel(X, P_logit, Q):
    T, N, F, L, A_ = X.shape
    pdim = L - K + 1

    Xp = X.transpose(0, 2, 4, 1, 3).reshape(T * F, A_, N * L)

    R, Zt, Sp = pl.pallas_call(
        functools.partial(_conv_kernel, pdim=pdim, n_batch=N, lanes=N * L),
        grid=(T * F,),
        in_specs=[
            pl.BlockSpec((K, A_, U), lambda b: (0, 0, 0)),
            pl.BlockSpec((1, A_), lambda b: (0, 0)),
            pl.BlockSpec((1, A_, N * L), lambda b: (b, 0, 0)),
        ],
        out_specs=[
            pl.BlockSpec((K, A_, U), lambda b: (0, 0, 0)),
            pl.BlockSpec((1, N, 1, U, pdim), lambda b: (b // F, 0, b % F, 0, 0)),
            pl.BlockSpec((1, 1, N, U), lambda b: (b // F, b % F, 0, 0)),
        ],
        out_shape=[
            jax.ShapeDtypeStruct((K, A_, U), jnp.float32),
            jax.ShapeDtypeStruct((T, N, F, U, pdim), jnp.float32),
            jax.ShapeDtypeStruct((T, F, N, U), jnp.float32),
        ],
        scratch_shapes=[
            pltpu.VMEM((NG, U, AP * GK), jnp.bfloat16),
        ],
    )(P_logit, Q.reshape(1, A_), Xp)

    return (R, Sp.transpose(0, 2, 1, 3), jnp.swapaxes(Zt, 3, 4))
